# baseline (speedup 1.0000x reference)
import jax
import jax.numpy as jnp
from jax.experimental import pallas as pl
from jax.experimental.pallas import tpu as pltpu

_CH = 1000


def _linear_kernel(h_hbm, w_ref, b_ref, o_hbm, ibuf, obuf, isem, osem):
    m = h_hbm.shape[0]
    n = m // _CH

    def load(i, slot):
        pltpu.make_async_copy(
            h_hbm.at[pl.ds(i * _CH, _CH), :], ibuf.at[slot], isem.at[slot]
        ).start()

    def store(i, slot):
        pltpu.make_async_copy(
            obuf.at[slot], o_hbm.at[pl.ds(i * _CH, _CH), :], osem.at[slot]
        ).start()

    load(0, 0)
    if n > 1:
        load(1, 1)
    for i in range(n):
        slot = i % 2
        pltpu.make_async_copy(
            h_hbm.at[pl.ds(i * _CH, _CH), :], ibuf.at[slot], isem.at[slot]
        ).wait()
        if i >= 2:
            pltpu.make_async_copy(
                obuf.at[slot], o_hbm.at[pl.ds((i - 2) * _CH, _CH), :], osem.at[slot]
            ).wait()
        obuf[slot] = jax.lax.dot_general(
            ibuf[slot], w_ref[...],
            dimension_numbers=(((1,), (1,)), ((), ())),
            preferred_element_type=jnp.float32,
        ) + b_ref[...]
        store(i, slot)
        if i + 2 < n:
            load(i + 2, slot)
    for i in (n - 2, n - 1):
        if i >= 0:
            slot = i % 2
            pltpu.make_async_copy(
                obuf.at[slot], o_hbm.at[pl.ds(i * _CH, _CH), :], osem.at[slot]
            ).wait()


def kernel(h, edge_index, edge_sh, edge_radial, n_atoms, W, b):
    del edge_index, edge_sh, edge_radial, n_atoms
    m, d = h.shape
    out = pl.pallas_call(
        _linear_kernel,
        in_specs=[
            pl.BlockSpec(memory_space=pl.ANY),
            pl.BlockSpec(memory_space=pltpu.VMEM),
            pl.BlockSpec(memory_space=pltpu.VMEM),
        ],
        out_specs=pl.BlockSpec(memory_space=pl.ANY),
        out_shape=jax.ShapeDtypeStruct((m, d), jnp.float32),
        scratch_shapes=[
            pltpu.VMEM((2, _CH, d), jnp.float32),
            pltpu.VMEM((2, _CH, d), jnp.float32),
            pltpu.SemaphoreType.DMA((2,)),
            pltpu.SemaphoreType.DMA((2,)),
        ],
    )(h, W, b.reshape(1, d))
    return out


# speedup vs baseline: 1.3231x; 1.3231x over previous
import jax
import jax.numpy as jnp
from jax.experimental import pallas as pl
from jax.experimental.pallas import tpu as pltpu

_CH = 1000


def _linear_kernel(h_hbm, w_ref, b_ref, o_hbm, ibuf, obuf, isem, osem):
    m = h_hbm.shape[0]
    n = m // _CH

    for i in range(n):
        pltpu.make_async_copy(
            h_hbm.at[pl.ds(i * _CH, _CH), :], ibuf.at[i], isem.at[i]
        ).start()
    for i in range(n):
        pltpu.make_async_copy(
            h_hbm.at[pl.ds(i * _CH, _CH), :], ibuf.at[i], isem.at[i]
        ).wait()
        obuf[i] = jax.lax.dot_general(
            ibuf[i], w_ref[...],
            dimension_numbers=(((1,), (1,)), ((), ())),
            preferred_element_type=jnp.float32,
        ) + b_ref[...]
        pltpu.make_async_copy(
            obuf.at[i], o_hbm.at[pl.ds(i * _CH, _CH), :], osem.at[i]
        ).start()
    for i in range(n):
        pltpu.make_async_copy(
            obuf.at[i], o_hbm.at[pl.ds(i * _CH, _CH), :], osem.at[i]
        ).wait()


def kernel(h, edge_index, edge_sh, edge_radial, n_atoms, W, b):
    del edge_index, edge_sh, edge_radial, n_atoms
    m, d = h.shape
    n = m // _CH
    out = pl.pallas_call(
        _linear_kernel,
        in_specs=[
            pl.BlockSpec(memory_space=pl.ANY),
            pl.BlockSpec(memory_space=pltpu.VMEM),
            pl.BlockSpec(memory_space=pltpu.VMEM),
        ],
        out_specs=pl.BlockSpec(memory_space=pl.ANY),
        out_shape=jax.ShapeDtypeStruct((m, d), jnp.float32),
        scratch_shapes=[
            pltpu.VMEM((n, _CH, d), jnp.float32),
            pltpu.VMEM((n, _CH, d), jnp.float32),
            pltpu.SemaphoreType.DMA((n,)),
            pltpu.SemaphoreType.DMA((n,)),
        ],
    )(h, W, b.reshape(1, d))
    return out


# speedup vs baseline: 1.7612x; 1.3312x over previous
import jax
import jax.numpy as jnp
from jax.experimental import pallas as pl
from jax.experimental.pallas import tpu as pltpu

_CH = 2000


def _linear_kernel(h_hbm, w_ref, b_ref, o_hbm, ibuf, obuf, isem, osem):
    m = h_hbm.shape[0]
    n = m // _CH

    for i in range(n):
        pltpu.make_async_copy(
            h_hbm.at[pl.ds(i * _CH, _CH), :], ibuf.at[i], isem.at[i]
        ).start()
    for i in range(n):
        pltpu.make_async_copy(
            h_hbm.at[pl.ds(i * _CH, _CH), :], ibuf.at[i], isem.at[i]
        ).wait()
        obuf[i] = jax.lax.dot_general(
            ibuf[i], w_ref[...],
            dimension_numbers=(((1,), (1,)), ((), ())),
            preferred_element_type=jnp.float32,
        ) + b_ref[...]
        pltpu.make_async_copy(
            obuf.at[i], o_hbm.at[pl.ds(i * _CH, _CH), :], osem.at[i]
        ).start()
    for i in range(n):
        pltpu.make_async_copy(
            obuf.at[i], o_hbm.at[pl.ds(i * _CH, _CH), :], osem.at[i]
        ).wait()


def kernel(h, edge_index, edge_sh, edge_radial, n_atoms, W, b):
    del edge_index, edge_sh, edge_radial, n_atoms
    m, d = h.shape
    n = m // _CH
    out = pl.pallas_call(
        _linear_kernel,
        in_specs=[
            pl.BlockSpec(memory_space=pl.ANY),
            pl.BlockSpec(memory_space=pltpu.VMEM),
            pl.BlockSpec(memory_space=pltpu.VMEM),
        ],
        out_specs=pl.BlockSpec(memory_space=pl.ANY),
        out_shape=jax.ShapeDtypeStruct((m, d), jnp.float32),
        scratch_shapes=[
            pltpu.VMEM((n, _CH, d), jnp.float32),
            pltpu.VMEM((n, _CH, d), jnp.float32),
            pltpu.SemaphoreType.DMA((n,)),
            pltpu.SemaphoreType.DMA((n,)),
        ],
    )(h, W, b.reshape(1, d))
    return out
